# Initial kernel scaffold; baseline (speedup 1.0000x reference)
import jax
import jax.numpy as jnp
from jax.experimental import pallas as pl


def kernel(x, cfx_x, y, W1, b1, W2, b2, Wf, bf):
    raise NotImplementedError("write your pallas kernel here")


# speedup vs baseline: 4.5231x; 4.5231x over previous
import jax
import jax.numpy as jnp
from jax.experimental import pallas as pl
from jax.experimental.pallas import tpu as pltpu

_EPS = 1e-08
_FAKE_INF = 10.0
_EPSILON = 0.01
_BIAS_EPSILON = 0.01

_BB = 256


def _fwd(xb, W1t, b1, W2t, aW2t, b2):
    om = jnp.dot(xb, W1t, preferred_element_type=jnp.float32) + b1
    r0 = _EPSILON * jnp.sum(jnp.abs(xb), axis=-1, keepdims=True) + _BIAS_EPSILON
    lb = jnp.maximum(om - r0, 0.0)
    ub = jnp.maximum(om + r0, 0.0)
    mu = 0.5 * (lb + ub)
    r = 0.5 * (ub - lb)
    om2 = jnp.dot(mu, W2t, preferred_element_type=jnp.float32) + b2
    or2 = (jnp.dot(r, aW2t, preferred_element_type=jnp.float32)
           + _EPSILON * jnp.sum(jnp.abs(mu) + r, axis=-1, keepdims=True)
           + _BIAS_EPSILON)
    return jnp.maximum(om2 - or2, 0.0), jnp.maximum(om2 + or2, 0.0)


def _bound(lb, ub, w, bsc):
    wlb = w - 2.0 * _EPSILON
    wub = w + 2.0 * _EPSILON
    p1 = lb * wlb
    p2 = ub * wlb
    left_lb = jnp.minimum(p1, p2)
    left_ub = jnp.maximum(p1, p2)
    q1 = lb * wub
    q2 = ub * wub
    right_lb = jnp.minimum(q1, q2)
    right_ub = jnp.maximum(q1, q2)
    ak = (right_lb - left_lb) / (4.0 * _EPSILON)
    ab = jnp.sum(left_lb - ak * wlb, axis=-1, keepdims=True) + (bsc - 2.0 * _BIAS_EPSILON)
    bk = (right_ub - left_ub) / (4.0 * _EPSILON)
    bb = jnp.sum(left_ub - bk * wlb, axis=-1, keepdims=True) + (bsc + 2.0 * _BIAS_EPSILON)
    return ak, ab, bk, bb


def _knap(k, k1, b, b1, wlb, wub):
    w_ret = jnp.where((k < 0) | ((k == 0) & (k1 < 0)), wlb, wub)
    ret = jnp.sum(w_ret * k1, axis=-1, keepdims=True) + b1
    t = jnp.sum(w_ret * k, axis=-1, keepdims=True) + b
    sv = jnp.where(k * k1 >= 0, 0.0, jnp.abs(k1) / (_EPS + jnp.abs(k)))
    dw = wub - wlb
    td = jnp.abs(k) * dw
    t1d = jnp.abs(k1) * dw
    pos = sv > 0
    zero = jnp.zeros_like(td)
    total_td = jnp.sum(jnp.where(pos, td, zero), axis=-1, keepdims=True)
    total_t1d = jnp.sum(jnp.where(pos, t1d, zero), axis=-1, keepdims=True)

    lo = jnp.zeros(t.shape, dtype=jnp.int32)
    hi = jnp.full(t.shape, 0x7F800000, dtype=jnp.int32)

    def body(_, carry):
        lo_, hi_ = carry
        mid = lo_ + (hi_ - lo_) // 2
        lam = jax.lax.bitcast_convert_type(mid, jnp.float32)
        fsum = jnp.sum(jnp.where(sv > lam, td, zero), axis=-1, keepdims=True)
        le = fsum <= t
        return jnp.where(le, lo_, mid), jnp.where(le, mid, hi_)

    lo, hi = jax.lax.fori_loop(0, 31, body, (lo, hi))
    lam_b = jax.lax.bitcast_convert_type(hi, jnp.float32)

    gt = sv > lam_b
    eq = sv == lam_b
    before_td = jnp.sum(jnp.where(gt, td, zero), axis=-1, keepdims=True)
    before_t1d = jnp.sum(jnp.where(gt, t1d, zero), axis=-1, keepdims=True)
    td_b = jnp.sum(jnp.where(eq, td, zero), axis=-1, keepdims=True)
    t1d_b = jnp.sum(jnp.where(eq, t1d, zero), axis=-1, keepdims=True)
    delta = t - before_td - td_b
    pct = jnp.clip(delta / (td_b + _EPS), -1.0, 0.0) + 1.0
    contrib = jnp.where(total_td <= t, total_t1d, before_t1d + t1d_b * pct)
    return jnp.where(t >= 0, ret + contrib, -_FAKE_INF)


def _block_body(x_ref, c_ref, y_ref, W1t_ref, b1_ref, W2t_ref, b2_ref,
                Wf_ref, bf_ref, ov_ref, oo_ref):
    W1t = W1t_ref[...]
    b1 = b1_ref[...]
    W2t = W2t_ref[...]
    aW2t = jnp.abs(W2t)
    b2 = b2_ref[...]
    w = Wf_ref[1:2, :] - Wf_ref[0:1, :]
    bsc = bf_ref[0:1, 1:2] - bf_ref[0:1, 0:1]

    elb, eub = _fwd(x_ref[...], W1t, b1, W2t, aW2t, b2)
    clb, cub = _fwd(c_ref[...], W1t, b1, W2t, aW2t, b2)

    aek, aeb, bek, beb = _bound(elb, eub, w, bsc)
    ack, acb, bck, bcb = _bound(clb, cub, w, bsc)

    wlb = w - 2.0 * _EPSILON
    wub = w + 2.0 * _EPSILON
    lbv = -_knap(-aek, -ack, -aeb, -acb, wlb, wub)
    ubv = _knap(bek, bck, beb, bcb, wlb, wub)

    yv = y_ref[...]
    ov_ref[...] = jnp.where(yv == 0,
                            (lbv <= 0.0).astype(jnp.int32),
                            (ubv >= 0.0).astype(jnp.int32))
    oo_ref[...] = jnp.where(yv == 0, lbv, ubv)


def kernel(x, cfx_x, y, W1, b1, W2, b2, Wf, bf):
    B, D = x.shape
    H = W1.shape[0]
    y2 = y.reshape(B, 1).astype(jnp.int32)
    W1t = W1.T
    W2t = W2.T
    b1r = b1.reshape(1, H)
    b2r = b2.reshape(1, H)
    bfr = bf.reshape(1, 2)

    grid = (B // _BB,)
    row = lambda i: (i, 0)
    rep = lambda i: (0, 0)
    valid_i, out_f = pl.pallas_call(
        _block_body,
        grid=grid,
        in_specs=[
            pl.BlockSpec((_BB, D), row),
            pl.BlockSpec((_BB, D), row),
            pl.BlockSpec((_BB, 1), row),
            pl.BlockSpec((D, H), rep),
            pl.BlockSpec((1, H), rep),
            pl.BlockSpec((H, H), rep),
            pl.BlockSpec((1, H), rep),
            pl.BlockSpec((2, H), rep),
            pl.BlockSpec((1, 2), rep),
        ],
        out_specs=[
            pl.BlockSpec((_BB, 1), row),
            pl.BlockSpec((_BB, 1), row),
        ],
        out_shape=[
            jax.ShapeDtypeStruct((B, 1), jnp.int32),
            jax.ShapeDtypeStruct((B, 1), jnp.float32),
        ],
        compiler_params=pltpu.CompilerParams(
            dimension_semantics=("arbitrary",),
        ),
    )(x, cfx_x, y2, W1t, b1r, W2t, b2r, Wf, bfr)
    return valid_i.reshape(B) != 0, out_f.reshape(B)


# speedup vs baseline: 26.1612x; 5.7839x over previous
import jax
import jax.numpy as jnp
from jax.experimental import pallas as pl
from jax.experimental.pallas import tpu as pltpu

_EPS = 1e-08
_FAKE_INF = 10.0
_EPSILON = 0.01
_BIAS_EPSILON = 0.01

_BB = 256


def _fwd(xb, W1t, b1, W2t, aW2t, b2):
    om = jnp.dot(xb, W1t, preferred_element_type=jnp.float32) + b1
    r0 = _EPSILON * jnp.sum(jnp.abs(xb), axis=-1, keepdims=True) + _BIAS_EPSILON
    lb = jnp.maximum(om - r0, 0.0)
    ub = jnp.maximum(om + r0, 0.0)
    mu = 0.5 * (lb + ub)
    r = 0.5 * (ub - lb)
    om2 = jnp.dot(mu, W2t, preferred_element_type=jnp.float32) + b2
    or2 = (jnp.dot(r, aW2t, preferred_element_type=jnp.float32)
           + _EPSILON * jnp.sum(jnp.abs(mu) + r, axis=-1, keepdims=True)
           + _BIAS_EPSILON)
    return jnp.maximum(om2 - or2, 0.0), jnp.maximum(om2 + or2, 0.0)


def _bound(lb, ub, w, bsc):
    wlb = w - 2.0 * _EPSILON
    wub = w + 2.0 * _EPSILON
    p1 = lb * wlb
    p2 = ub * wlb
    left_lb = jnp.minimum(p1, p2)
    left_ub = jnp.maximum(p1, p2)
    q1 = lb * wub
    q2 = ub * wub
    right_lb = jnp.minimum(q1, q2)
    right_ub = jnp.maximum(q1, q2)
    ak = (right_lb - left_lb) / (4.0 * _EPSILON)
    ab = jnp.sum(left_lb - ak * wlb, axis=-1, keepdims=True) + (bsc - 2.0 * _BIAS_EPSILON)
    bk = (right_ub - left_ub) / (4.0 * _EPSILON)
    bb = jnp.sum(left_ub - bk * wlb, axis=-1, keepdims=True) + (bsc + 2.0 * _BIAS_EPSILON)
    return ak, ab, bk, bb


def _reduced(k, k1, b, b1, wlb, wub):
    w_ret = jnp.where((k < 0) | ((k == 0) & (k1 < 0)), wlb, wub)
    ret = jnp.sum(w_ret * k1, axis=-1, keepdims=True) + b1
    t = jnp.sum(w_ret * k, axis=-1, keepdims=True) + b
    return jnp.where(t >= 0, ret, -_FAKE_INF)


def _block_body(x_ref, c_ref, y_ref, W1t_ref, b1_ref, W2t_ref, b2_ref,
                Wf_ref, bf_ref, ov_ref, oo_ref):
    W1t = W1t_ref[...]
    b1 = b1_ref[...]
    W2t = W2t_ref[...]
    aW2t = jnp.abs(W2t)
    b2 = b2_ref[...]
    w = Wf_ref[1:2, :] - Wf_ref[0:1, :]
    bsc = bf_ref[0:1, 1:2] - bf_ref[0:1, 0:1]

    elb, eub = _fwd(x_ref[...], W1t, b1, W2t, aW2t, b2)
    clb, cub = _fwd(c_ref[...], W1t, b1, W2t, aW2t, b2)

    aek, aeb, bek, beb = _bound(elb, eub, w, bsc)
    ack, acb, bck, bcb = _bound(clb, cub, w, bsc)

    wlb = w - 2.0 * _EPSILON
    wub = w + 2.0 * _EPSILON
    lbv = -_reduced(-aek, -ack, -aeb, -acb, wlb, wub)
    ubv = _reduced(bek, bck, beb, bcb, wlb, wub)

    yv = y_ref[...]
    ov_ref[...] = jnp.where(yv == 0,
                            (lbv <= 0.0).astype(jnp.int32),
                            (ubv >= 0.0).astype(jnp.int32))
    oo_ref[...] = jnp.where(yv == 0, lbv, ubv)


def kernel(x, cfx_x, y, W1, b1, W2, b2, Wf, bf):
    B, D = x.shape
    H = W1.shape[0]
    y2 = y.reshape(B, 1).astype(jnp.int32)
    W1t = W1.T
    W2t = W2.T
    b1r = b1.reshape(1, H)
    b2r = b2.reshape(1, H)
    bfr = bf.reshape(1, 2)

    grid = (B // _BB,)
    row = lambda i: (i, 0)
    rep = lambda i: (0, 0)
    valid_i, out_f = pl.pallas_call(
        _block_body,
        grid=grid,
        in_specs=[
            pl.BlockSpec((_BB, D), row),
            pl.BlockSpec((_BB, D), row),
            pl.BlockSpec((_BB, 1), row),
            pl.BlockSpec((D, H), rep),
            pl.BlockSpec((1, H), rep),
            pl.BlockSpec((H, H), rep),
            pl.BlockSpec((1, H), rep),
            pl.BlockSpec((2, H), rep),
            pl.BlockSpec((1, 2), rep),
        ],
        out_specs=[
            pl.BlockSpec((_BB, 1), row),
            pl.BlockSpec((_BB, 1), row),
        ],
        out_shape=[
            jax.ShapeDtypeStruct((B, 1), jnp.int32),
            jax.ShapeDtypeStruct((B, 1), jnp.float32),
        ],
        compiler_params=pltpu.CompilerParams(
            dimension_semantics=("arbitrary",),
        ),
    )(x, cfx_x, y2, W1t, b1r, W2t, b2r, Wf, bfr)
    return valid_i.reshape(B) != 0, out_f.reshape(B)


# speedup vs baseline: 33.2293x; 1.2702x over previous
import jax
import jax.numpy as jnp
from jax.experimental import pallas as pl
from jax.experimental.pallas import tpu as pltpu

_EPS = 1e-08
_FAKE_INF = 10.0
_EPSILON = 0.01
_BIAS_EPSILON = 0.01

_BB = 256


def _fwd(xb, W1t, b1, W2t, aW2t, b2):
    om = jnp.dot(xb, W1t, preferred_element_type=jnp.float32) + b1
    r0 = _EPSILON * jnp.sum(jnp.abs(xb), axis=-1, keepdims=True) + _BIAS_EPSILON
    lb = jnp.maximum(om - r0, 0.0)
    ub = jnp.maximum(om + r0, 0.0)
    mu = 0.5 * (lb + ub)
    r = 0.5 * (ub - lb)
    om2 = jnp.dot(mu, W2t, preferred_element_type=jnp.float32) + b2
    or2 = (jnp.dot(r, aW2t, preferred_element_type=jnp.float32)
           + _EPSILON * jnp.sum(mu + r, axis=-1, keepdims=True)
           + _BIAS_EPSILON)
    return jnp.maximum(om2 - or2, 0.0), jnp.maximum(om2 + or2, 0.0)


def _sums(lb, ub, wlb, wub, wlb_pos, wub_pos):
    left_lb = wlb * jnp.where(wlb_pos, lb, ub)
    right_ub = wub * jnp.where(wub_pos, ub, lb)
    return (jnp.sum(left_lb, axis=-1, keepdims=True),
            jnp.sum(right_ub, axis=-1, keepdims=True))


def _block_body(x_ref, c_ref, y_ref, W1t_ref, b1_ref, W2t_ref, b2_ref,
                Wf_ref, bf_ref, ov_ref, oo_ref):
    W1t = W1t_ref[...]
    b1 = b1_ref[...]
    W2t = W2t_ref[...]
    aW2t = jnp.abs(W2t)
    b2 = b2_ref[...]
    w = Wf_ref[1:2, :] - Wf_ref[0:1, :]
    bsc = bf_ref[0:1, 1:2] - bf_ref[0:1, 0:1]
    wlb = w - 2.0 * _EPSILON
    wub = w + 2.0 * _EPSILON
    wlb_pos = wlb >= 0
    wub_pos = wub >= 0

    elb, eub = _fwd(x_ref[...], W1t, b1, W2t, aW2t, b2)
    clb, cub = _fwd(c_ref[...], W1t, b1, W2t, aW2t, b2)

    a_e, t_e = _sums(elb, eub, wlb, wub, wlb_pos, wub_pos)
    a_c, t_c = _sums(clb, cub, wlb, wub, wlb_pos, wub_pos)

    off_lo = bsc - 2.0 * _BIAS_EPSILON
    off_hi = bsc + 2.0 * _BIAS_EPSILON
    lbv = jnp.where(a_e + off_lo <= 0, a_c + off_lo, _FAKE_INF)
    ubv = jnp.where(t_e + off_hi >= 0, t_c + off_hi, -_FAKE_INF)

    yv = y_ref[...]
    ov_ref[...] = jnp.where(yv == 0,
                            (lbv <= 0.0).astype(jnp.int32),
                            (ubv >= 0.0).astype(jnp.int32))
    oo_ref[...] = jnp.where(yv == 0, lbv, ubv)


def kernel(x, cfx_x, y, W1, b1, W2, b2, Wf, bf):
    B, D = x.shape
    H = W1.shape[0]
    y2 = y.reshape(B, 1).astype(jnp.int32)
    W1t = W1.T
    W2t = W2.T
    b1r = b1.reshape(1, H)
    b2r = b2.reshape(1, H)
    bfr = bf.reshape(1, 2)

    grid = (B // _BB,)
    row = lambda i: (i, 0)
    rep = lambda i: (0, 0)
    valid_i, out_f = pl.pallas_call(
        _block_body,
        grid=grid,
        in_specs=[
            pl.BlockSpec((_BB, D), row),
            pl.BlockSpec((_BB, D), row),
            pl.BlockSpec((_BB, 1), row),
            pl.BlockSpec((D, H), rep),
            pl.BlockSpec((1, H), rep),
            pl.BlockSpec((H, H), rep),
            pl.BlockSpec((1, H), rep),
            pl.BlockSpec((2, H), rep),
            pl.BlockSpec((1, 2), rep),
        ],
        out_specs=[
            pl.BlockSpec((_BB, 1), row),
            pl.BlockSpec((_BB, 1), row),
        ],
        out_shape=[
            jax.ShapeDtypeStruct((B, 1), jnp.int32),
            jax.ShapeDtypeStruct((B, 1), jnp.float32),
        ],
        compiler_params=pltpu.CompilerParams(
            dimension_semantics=("arbitrary",),
        ),
    )(x, cfx_x, y2, W1t, b1r, W2t, b2r, Wf, bfr)
    return valid_i.reshape(B) != 0, out_f.reshape(B)


# speedup vs baseline: 34.3641x; 1.0342x over previous
import jax
import jax.numpy as jnp
from jax.experimental import pallas as pl
from jax.experimental.pallas import tpu as pltpu

_EPS = 1e-08
_FAKE_INF = 10.0
_EPSILON = 0.01
_BIAS_EPSILON = 0.01

_BB = 512


def _fwd(xb, W1t, b1, W2t, aW2t, b2):
    om = jnp.dot(xb, W1t, preferred_element_type=jnp.float32) + b1
    r0 = _EPSILON * jnp.sum(jnp.abs(xb), axis=-1, keepdims=True) + _BIAS_EPSILON
    lb = jnp.maximum(om - r0, 0.0)
    ub = jnp.maximum(om + r0, 0.0)
    mu = 0.5 * (lb + ub)
    r = 0.5 * (ub - lb)
    om2 = jnp.dot(mu, W2t, preferred_element_type=jnp.float32) + b2
    or2 = (jnp.dot(r, aW2t, preferred_element_type=jnp.float32)
           + _EPSILON * jnp.sum(mu + r, axis=-1, keepdims=True)
           + _BIAS_EPSILON)
    return jnp.maximum(om2 - or2, 0.0), jnp.maximum(om2 + or2, 0.0)


def _sums(lb, ub, wlb, wub, wlb_pos, wub_pos):
    left_lb = wlb * jnp.where(wlb_pos, lb, ub)
    right_ub = wub * jnp.where(wub_pos, ub, lb)
    return (jnp.sum(left_lb, axis=-1, keepdims=True),
            jnp.sum(right_ub, axis=-1, keepdims=True))


def _block_body(x_ref, c_ref, y_ref, W1t_ref, b1_ref, W2t_ref, b2_ref,
                Wf_ref, bf_ref, ov_ref, oo_ref):
    W1t = W1t_ref[...]
    b1 = b1_ref[...]
    W2t = W2t_ref[...]
    aW2t = jnp.abs(W2t)
    b2 = b2_ref[...]
    w = Wf_ref[1:2, :] - Wf_ref[0:1, :]
    bsc = bf_ref[0:1, 1:2] - bf_ref[0:1, 0:1]
    wlb = w - 2.0 * _EPSILON
    wub = w + 2.0 * _EPSILON
    wlb_pos = wlb >= 0
    wub_pos = wub >= 0

    elb, eub = _fwd(x_ref[...], W1t, b1, W2t, aW2t, b2)
    clb, cub = _fwd(c_ref[...], W1t, b1, W2t, aW2t, b2)

    a_e, t_e = _sums(elb, eub, wlb, wub, wlb_pos, wub_pos)
    a_c, t_c = _sums(clb, cub, wlb, wub, wlb_pos, wub_pos)

    off_lo = bsc - 2.0 * _BIAS_EPSILON
    off_hi = bsc + 2.0 * _BIAS_EPSILON
    lbv = jnp.where(a_e + off_lo <= 0, a_c + off_lo, _FAKE_INF)
    ubv = jnp.where(t_e + off_hi >= 0, t_c + off_hi, -_FAKE_INF)

    yv = y_ref[...]
    ov_ref[...] = jnp.where(yv == 0,
                            (lbv <= 0.0).astype(jnp.int32),
                            (ubv >= 0.0).astype(jnp.int32))
    oo_ref[...] = jnp.where(yv == 0, lbv, ubv)


def kernel(x, cfx_x, y, W1, b1, W2, b2, Wf, bf):
    B, D = x.shape
    H = W1.shape[0]
    y2 = y.reshape(B, 1).astype(jnp.int32)
    W1t = W1.T
    W2t = W2.T
    b1r = b1.reshape(1, H)
    b2r = b2.reshape(1, H)
    bfr = bf.reshape(1, 2)

    grid = (B // _BB,)
    row = lambda i: (i, 0)
    rep = lambda i: (0, 0)
    valid_i, out_f = pl.pallas_call(
        _block_body,
        grid=grid,
        in_specs=[
            pl.BlockSpec((_BB, D), row),
            pl.BlockSpec((_BB, D), row),
            pl.BlockSpec((_BB, 1), row),
            pl.BlockSpec((D, H), rep),
            pl.BlockSpec((1, H), rep),
            pl.BlockSpec((H, H), rep),
            pl.BlockSpec((1, H), rep),
            pl.BlockSpec((2, H), rep),
            pl.BlockSpec((1, 2), rep),
        ],
        out_specs=[
            pl.BlockSpec((_BB, 1), row),
            pl.BlockSpec((_BB, 1), row),
        ],
        out_shape=[
            jax.ShapeDtypeStruct((B, 1), jnp.int32),
            jax.ShapeDtypeStruct((B, 1), jnp.float32),
        ],
        compiler_params=pltpu.CompilerParams(
            dimension_semantics=("arbitrary",),
        ),
    )(x, cfx_x, y2, W1t, b1r, W2t, b2r, Wf, bfr)
    return valid_i.reshape(B) != 0, out_f.reshape(B)
